# baseline (speedup 1.0000x reference)
import functools
import math

import jax
import jax.numpy as jnp
from jax import lax
from jax.experimental import pallas as pl
from jax.experimental.pallas import tpu as pltpu, tpu_sc as plsc

N = 10000
E = 320000
D = 128
NB = 10
RN = 50
C_S = math.sin(math.pi / 8)
C_X = math.cos(math.pi / 8)

NC = 2
NS = 16
NW = NC * NS
EPT = E // NW
K = 80
NBLK = EPT // K
ROWS_PER_SUB = 624
ROWS_TAIL = N - (NS - 1) * ROWS_PER_SUB - ROWS_PER_SUB

_PREC = jax.lax.Precision.HIGHEST

_PERM = [32 * q + t for q in range(4) for t in range(16)]
_PERM = _PERM + [p + 16 for p in _PERM]

_BE = 4000


def _weights_body(es_ref, ea_ref, f1_ref, f2_ref, *outs):
    es = es_ref[...]
    ea = ea_ref[...]
    z = jnp.dot(es, f1_ref[...], preferred_element_type=jnp.float32)
    h = z * jax.nn.sigmoid(z)
    w = jnp.dot(h, f2_ref[...], preferred_element_type=jnp.float32)
    for i, o in enumerate(outs):
        wi = w[:, i * D:(i + 1) * D] * ea
        ua = jax.lax.bitcast_convert_type(wi[:, :D // 2], jnp.uint32)
        ub = jax.lax.bitcast_convert_type(wi[:, D // 2:], jnp.uint32)
        rnd = jnp.uint32(0x8000)
        o[...] = (((ub + rnd) & jnp.uint32(0xFFFF0000))
                  | ((ua + rnd) >> 16))


def _edge_weights(es, ea, f1s, f2s):
    nl = len(f1s)
    f1cat = jnp.concatenate(f1s, axis=1)
    f2bd = jax.scipy.linalg.block_diag(*[f2[:, _PERM] for f2 in f2s])
    grid = (E // _BE,)
    w_spec = lambda shp: pl.BlockSpec(shp, lambda i: (0, 0))
    return pl.pallas_call(
        _weights_body,
        grid=grid,
        in_specs=[
            pl.BlockSpec((_BE, NB), lambda i: (i, 0)),
            pl.BlockSpec((_BE, 1), lambda i: (i, 0)),
            w_spec((NB, nl * RN)),
            w_spec((nl * RN, nl * D)),
        ],
        out_specs=[pl.BlockSpec((_BE, D // 2), lambda i: (i, 0))] * nl,
        out_shape=[jax.ShapeDtypeStruct((E, D // 2), jnp.uint32)] * nl,
    )(es, ea, f1cat, f2bd)


_BN = 2000


def _pre_body(x_ref, a_ref, wsc_ref, wl1_ref, s_ref, xl_ref):
    xa = x_ref[...] * a_ref[...]
    s_ref[...] = jnp.dot(xa, wsc_ref[...], preferred_element_type=jnp.float32,
                         precision=_PREC)
    xl_ref[...] = jnp.dot(xa, wl1_ref[...], preferred_element_type=jnp.float32,
                          precision=_PREC)


def _pre(x, a, wsc, wl1):
    grid = (N // _BN,)
    return pl.pallas_call(
        _pre_body,
        grid=grid,
        in_specs=[
            pl.BlockSpec((_BN, D), lambda i: (i, 0)),
            pl.BlockSpec((_BN, 1), lambda i: (i, 0)),
            pl.BlockSpec((D, D), lambda i: (0, 0)),
            pl.BlockSpec((D, D), lambda i: (0, 0)),
        ],
        out_specs=[pl.BlockSpec((_BN, D), lambda i: (i, 0))] * 2,
        out_shape=[jax.ShapeDtypeStruct((N, D), jnp.float32)] * 2,
    )(x, a, wsc, wl1)


def _post_body(a0_ref, a1_ref, a_ref, s_ref, wl2_ref, o_ref, *, act):
    agg = (a0_ref[...] + a1_ref[...]) * a_ref[...]
    y = C_S * s_ref[...] + C_X * jnp.dot(
        agg, wl2_ref[...], preferred_element_type=jnp.float32, precision=_PREC)
    if act:
        y = y * jax.nn.sigmoid(y)
    o_ref[...] = y


def _post(agg0, agg1, a, s, wl2, act):
    grid = (N // _BN,)
    return pl.pallas_call(
        functools.partial(_post_body, act=act),
        grid=grid,
        in_specs=[
            pl.BlockSpec((_BN, D), lambda i: (i, 0)),
            pl.BlockSpec((_BN, D), lambda i: (i, 0)),
            pl.BlockSpec((_BN, 1), lambda i: (i, 0)),
            pl.BlockSpec((_BN, D), lambda i: (i, 0)),
            pl.BlockSpec((D, D), lambda i: (0, 0)),
        ],
        out_specs=pl.BlockSpec((_BN, D), lambda i: (i, 0)),
        out_shape=jax.ShapeDtypeStruct((N, D), jnp.float32),
    )(agg0, agg1, a, s, wl2)


def _sc_body(xl_hbm, src_hbm, dst_hbm, w_hbm, zero_hbm, out_hbm,
             sv0, dv0, gv0, wv0, sv1, dv1, gv1, wv1, agg_sh,
             lsem0, gsem0, ssem0, lsem1, gsem1, ssem1):
    c = lax.axis_index("c")
    s = lax.axis_index("s")
    wid = c * NS + s
    r0 = s * ROWS_PER_SUB
    rtail = NS * ROWS_PER_SUB
    pltpu.sync_copy(zero_hbm.at[pl.ds(r0, ROWS_PER_SUB)],
                    agg_sh.at[pl.ds(r0, ROWS_PER_SUB)])

    @pl.when(s == NS - 1)
    def _():
        pltpu.sync_copy(zero_hbm.at[pl.ds(rtail, ROWS_TAIL)],
                        agg_sh.at[pl.ds(rtail, ROWS_TAIL)])

    plsc.subcore_barrier()

    ebase = wid * EPT
    slots = ((sv0, dv0, gv0, wv0, lsem0, gsem0, ssem0),
             (sv1, dv1, gv1, wv1, lsem1, gsem1, ssem1))

    def lin_start(j, sl):
        sv, dv, gv, wv, lsem, gsem, ssem = sl
        b = ebase + j * K
        pltpu.async_copy(src_hbm.at[pl.ds(b, K)], sv, lsem)
        pltpu.async_copy(dst_hbm.at[pl.ds(b, K)], dv, lsem)
        pltpu.async_copy(w_hbm.at[pl.ds(b, K)], wv, lsem)

    def lin_wait(sl):
        sv, dv, gv, wv, lsem, gsem, ssem = sl
        pltpu.make_async_copy(src_hbm.at[pl.ds(0, K)], sv, lsem).wait()
        pltpu.make_async_copy(dst_hbm.at[pl.ds(0, K)], dv, lsem).wait()
        pltpu.make_async_copy(w_hbm.at[pl.ds(0, K)], wv, lsem).wait()

    def gather_start(sl):
        sv, dv, gv, wv, lsem, gsem, ssem = sl
        pltpu.async_copy(xl_hbm.at[sv], gv, gsem)

    def gather_wait(sl):
        sv, dv, gv, wv, lsem, gsem, ssem = sl
        pltpu.make_async_copy(xl_hbm.at[sv], gv, gsem).wait()

    def mul(sl):
        sv, dv, gv, wv, lsem, gsem, ssem = sl

        @plsc.parallel_loop(0, K, 1, unroll=4)
        def _mrow(e):
            for q in range(D // 32):
                uw = wv[e, pl.ds(q * 16, 16)]
                hi = jnp.uint32(0xFFFF0000)
                wa = jax.lax.bitcast_convert_type(uw << 16, jnp.float32)
                wb = jax.lax.bitcast_convert_type(uw & hi, jnp.float32)
                sl0 = pl.ds(q * 32, 16)
                sl1 = pl.ds(q * 32 + 16, 16)
                gv[e, sl0] = gv[e, sl0] * wa
                gv[e, sl1] = gv[e, sl1] * wb

    def scat_start(sl):
        sv, dv, gv, wv, lsem, gsem, ssem = sl
        pltpu.async_copy(gv, agg_sh.at[dv], ssem, add=True)

    def scat_wait(sl):
        sv, dv, gv, wv, lsem, gsem, ssem = sl
        pltpu.make_async_copy(gv, agg_sh.at[dv], ssem).wait()

    def step(j, sl_cur, sl_nxt, has_next, has_next2):
        gather_wait(sl_cur)
        mul(sl_cur)
        scat_start(sl_cur)
        if has_next:
            lin_wait(sl_nxt)
            gather_start(sl_nxt)
        scat_wait(sl_cur)
        if has_next2:
            lin_start(j + 2, sl_cur)

    lin_start(0, slots[0])
    lin_wait(slots[0])
    gather_start(slots[0])
    lin_start(1, slots[1])

    def pair(t, carry):
        j = 2 * t
        step(j, slots[0], slots[1], True, True)
        step(j + 1, slots[1], slots[0], True, True)
        return carry

    lax.fori_loop(0, (NBLK - 3) // 2, pair, 0)
    step(NBLK - 3, slots[0], slots[1], True, True)
    step(NBLK - 2, slots[1], slots[0], True, False)
    step(NBLK - 1, slots[0], slots[1], False, False)

    plsc.subcore_barrier()
    pltpu.sync_copy(agg_sh.at[pl.ds(r0, ROWS_PER_SUB)],
                    out_hbm.at[pl.ds(c * N + r0, ROWS_PER_SUB)])

    @pl.when(s == NS - 1)
    def _():
        pltpu.sync_copy(agg_sh.at[pl.ds(rtail, ROWS_TAIL)],
                        out_hbm.at[pl.ds(c * N + rtail, ROWS_TAIL)])


_sc_scatter = functools.partial(
    pl.kernel,
    out_type=jax.ShapeDtypeStruct((2 * N, D), jnp.float32),
    mesh=plsc.VectorSubcoreMesh(core_axis_name="c", subcore_axis_name="s",
                                num_cores=NC, num_subcores=NS),
    scratch_types=[
        pltpu.VMEM((K,), jnp.int32),
        pltpu.VMEM((K,), jnp.int32),
        pltpu.VMEM((K, D), jnp.float32),
        pltpu.VMEM((K, D // 2), jnp.uint32),
        pltpu.VMEM((K,), jnp.int32),
        pltpu.VMEM((K,), jnp.int32),
        pltpu.VMEM((K, D), jnp.float32),
        pltpu.VMEM((K, D // 2), jnp.uint32),
        pltpu.VMEM_SHARED((N, D), jnp.float32),
        pltpu.SemaphoreType.DMA,
        pltpu.SemaphoreType.DMA,
        pltpu.SemaphoreType.DMA,
        pltpu.SemaphoreType.DMA,
        pltpu.SemaphoreType.DMA,
        pltpu.SemaphoreType.DMA,
    ],
)(_sc_body)


def kernel(node_input, node_attr, edge_index, edge_attr, edge_scalars,
           W_sc_0, W_lin1_0, W_fc1_0, W_fc2_0, W_lin2_0,
           W_sc_1, W_lin1_1, W_fc1_1, W_fc2_1, W_lin2_1,
           W_sc_2, W_lin1_2, W_fc1_2, W_fc2_2, W_lin2_2):
    src = edge_index[0]
    dst = edge_index[1]

    inv_d = 1.0 / math.sqrt(D)
    f1_scale = 1.0 / math.sqrt(NB)
    f2_scale = 1.0 / (math.sqrt(RN) * math.sqrt(32.0))

    f1s = [W_fc1_0 * f1_scale, W_fc1_1 * f1_scale, W_fc1_2 * f1_scale]
    f2s = [W_fc2_0 * f2_scale, W_fc2_1 * f2_scale, W_fc2_2 * f2_scale]
    wscs = [W_sc_0 * inv_d, W_sc_1 * inv_d, W_sc_2 * inv_d]
    wl1s = [W_lin1_0 * inv_d, W_lin1_1 * inv_d, W_lin1_2 * inv_d]
    wl2s = [W_lin2_0 * inv_d, W_lin2_1 * inv_d, W_lin2_2 * inv_d]

    zeros = jnp.zeros((N, D), jnp.float32)

    (w0,) = _edge_weights(edge_scalars, edge_attr, f1s[:1], f2s[:1])
    wfull = [w0, None, None]

    x = node_input
    for i in range(3):
        s, xl = _pre(x, node_attr, wscs[i], wl1s[i])
        aggp = _sc_scatter(xl, src, dst, wfull[i], zeros)
        if i < 2:
            (wfull[i + 1],) = _edge_weights(edge_scalars, edge_attr,
                                            f1s[i + 1:i + 2],
                                            f2s[i + 1:i + 2])
        x = _post(aggp[:N], aggp[N:], node_attr, s, wl2s[i], act=(i < 2))
    return x


# speedup vs baseline: 1.0457x; 1.0457x over previous
import functools
import math

import jax
import jax.numpy as jnp
from jax import lax
from jax.experimental import pallas as pl
from jax.experimental.pallas import tpu as pltpu, tpu_sc as plsc

N = 10000
E = 320000
D = 128
NB = 10
RN = 50
C_S = math.sin(math.pi / 8)
C_X = math.cos(math.pi / 8)

NC = 2
NS = 16
NW = NC * NS
EPT = E // NW
K = 80
NBLK = EPT // K
ROWS_PER_SUB = 624
ROWS_TAIL = N - (NS - 1) * ROWS_PER_SUB - ROWS_PER_SUB

_PREC = jax.lax.Precision.HIGHEST

_PERM = [32 * q + t for q in range(4) for t in range(16)]
_PERM = _PERM + [p + 16 for p in _PERM]

_BE = 4000


def _weights_body(es_ref, ea_ref, f1_ref, f2_ref, *outs):
    es = es_ref[...]
    ea = ea_ref[...]
    z = jnp.dot(es, f1_ref[...], preferred_element_type=jnp.float32)
    h = z * jax.nn.sigmoid(z)
    w = jnp.dot(h, f2_ref[...], preferred_element_type=jnp.float32)
    for i, o in enumerate(outs):
        wi = w[:, i * D:(i + 1) * D] * ea
        ua = jax.lax.bitcast_convert_type(wi[:, :D // 2], jnp.uint32)
        ub = jax.lax.bitcast_convert_type(wi[:, D // 2:], jnp.uint32)
        rnd = jnp.uint32(0x8000)
        o[...] = (((ub + rnd) & jnp.uint32(0xFFFF0000))
                  | ((ua + rnd) >> 16))


def _edge_weights(es, ea, f1s, f2s):
    nl = len(f1s)
    f1cat = jnp.concatenate(f1s, axis=1)
    f2bd = jax.scipy.linalg.block_diag(*[f2[:, _PERM] for f2 in f2s])
    grid = (E // _BE,)
    w_spec = lambda shp: pl.BlockSpec(shp, lambda i: (0, 0))
    return pl.pallas_call(
        _weights_body,
        grid=grid,
        in_specs=[
            pl.BlockSpec((_BE, NB), lambda i: (i, 0)),
            pl.BlockSpec((_BE, 1), lambda i: (i, 0)),
            w_spec((NB, nl * RN)),
            w_spec((nl * RN, nl * D)),
        ],
        out_specs=[pl.BlockSpec((_BE, D // 2), lambda i: (i, 0))] * nl,
        out_shape=[jax.ShapeDtypeStruct((E, D // 2), jnp.uint32)] * nl,
    )(es, ea, f1cat, f2bd)


_BN = 2000


def _pre_body(x_ref, a_ref, wsc_ref, wl1_ref, s_ref, xl_ref):
    xa = x_ref[...] * a_ref[...]
    s_ref[...] = jnp.dot(xa, wsc_ref[...], preferred_element_type=jnp.float32,
                         precision=_PREC)
    xl_ref[...] = jnp.dot(xa, wl1_ref[...], preferred_element_type=jnp.float32,
                          precision=_PREC)


def _pre(x, a, wsc, wl1):
    grid = (N // _BN,)
    return pl.pallas_call(
        _pre_body,
        grid=grid,
        in_specs=[
            pl.BlockSpec((_BN, D), lambda i: (i, 0)),
            pl.BlockSpec((_BN, 1), lambda i: (i, 0)),
            pl.BlockSpec((D, D), lambda i: (0, 0)),
            pl.BlockSpec((D, D), lambda i: (0, 0)),
        ],
        out_specs=[pl.BlockSpec((_BN, D), lambda i: (i, 0))] * 2,
        out_shape=[jax.ShapeDtypeStruct((N, D), jnp.float32)] * 2,
    )(x, a, wsc, wl1)


def _post_body(a0_ref, a1_ref, a_ref, s_ref, wl2_ref, o_ref, *, act):
    agg = (a0_ref[...] + a1_ref[...]) * a_ref[...]
    y = C_S * s_ref[...] + C_X * jnp.dot(
        agg, wl2_ref[...], preferred_element_type=jnp.float32, precision=_PREC)
    if act:
        y = y * jax.nn.sigmoid(y)
    o_ref[...] = y


def _postpre_body(a0_ref, a1_ref, a_ref, s_ref, wl2_ref, wsc_ref, wl1_ref,
                  sn_ref, xln_ref):
    agg = (a0_ref[...] + a1_ref[...]) * a_ref[...]
    y = C_S * s_ref[...] + C_X * jnp.dot(
        agg, wl2_ref[...], preferred_element_type=jnp.float32, precision=_PREC)
    y = y * jax.nn.sigmoid(y)
    ya = y * a_ref[...]
    sn_ref[...] = jnp.dot(ya, wsc_ref[...], preferred_element_type=jnp.float32,
                          precision=_PREC)
    xln_ref[...] = jnp.dot(ya, wl1_ref[...],
                           preferred_element_type=jnp.float32, precision=_PREC)


def _postpre(agg0, agg1, a, s, wl2, wsc_n, wl1_n):
    grid = (N // _BN,)
    return pl.pallas_call(
        _postpre_body,
        grid=grid,
        in_specs=[
            pl.BlockSpec((_BN, D), lambda i: (i, 0)),
            pl.BlockSpec((_BN, D), lambda i: (i, 0)),
            pl.BlockSpec((_BN, 1), lambda i: (i, 0)),
            pl.BlockSpec((_BN, D), lambda i: (i, 0)),
            pl.BlockSpec((D, D), lambda i: (0, 0)),
            pl.BlockSpec((D, D), lambda i: (0, 0)),
            pl.BlockSpec((D, D), lambda i: (0, 0)),
        ],
        out_specs=[pl.BlockSpec((_BN, D), lambda i: (i, 0))] * 2,
        out_shape=[jax.ShapeDtypeStruct((N, D), jnp.float32)] * 2,
    )(agg0, agg1, a, s, wl2, wsc_n, wl1_n)


def _post(agg0, agg1, a, s, wl2, act):
    grid = (N // _BN,)
    return pl.pallas_call(
        functools.partial(_post_body, act=act),
        grid=grid,
        in_specs=[
            pl.BlockSpec((_BN, D), lambda i: (i, 0)),
            pl.BlockSpec((_BN, D), lambda i: (i, 0)),
            pl.BlockSpec((_BN, 1), lambda i: (i, 0)),
            pl.BlockSpec((_BN, D), lambda i: (i, 0)),
            pl.BlockSpec((D, D), lambda i: (0, 0)),
        ],
        out_specs=pl.BlockSpec((_BN, D), lambda i: (i, 0)),
        out_shape=jax.ShapeDtypeStruct((N, D), jnp.float32),
    )(agg0, agg1, a, s, wl2)


def _sc_body(xl_hbm, src_hbm, dst_hbm, w_hbm, zero_hbm, out_hbm,
             sv0, dv0, gv0, wv0, sv1, dv1, gv1, wv1, agg_sh,
             lsem0, gsem0, ssem0, lsem1, gsem1, ssem1):
    c = lax.axis_index("c")
    s = lax.axis_index("s")
    wid = c * NS + s
    r0 = s * ROWS_PER_SUB
    rtail = NS * ROWS_PER_SUB
    pltpu.sync_copy(zero_hbm.at[pl.ds(r0, ROWS_PER_SUB)],
                    agg_sh.at[pl.ds(r0, ROWS_PER_SUB)])

    @pl.when(s == NS - 1)
    def _():
        pltpu.sync_copy(zero_hbm.at[pl.ds(rtail, ROWS_TAIL)],
                        agg_sh.at[pl.ds(rtail, ROWS_TAIL)])

    plsc.subcore_barrier()

    ebase = wid * EPT
    slots = ((sv0, dv0, gv0, wv0, lsem0, gsem0, ssem0),
             (sv1, dv1, gv1, wv1, lsem1, gsem1, ssem1))

    def lin_start(j, sl):
        sv, dv, gv, wv, lsem, gsem, ssem = sl
        b = ebase + j * K
        pltpu.async_copy(src_hbm.at[pl.ds(b, K)], sv, lsem)
        pltpu.async_copy(dst_hbm.at[pl.ds(b, K)], dv, lsem)
        pltpu.async_copy(w_hbm.at[pl.ds(b, K)], wv, lsem)

    def lin_wait(sl):
        sv, dv, gv, wv, lsem, gsem, ssem = sl
        pltpu.make_async_copy(src_hbm.at[pl.ds(0, K)], sv, lsem).wait()
        pltpu.make_async_copy(dst_hbm.at[pl.ds(0, K)], dv, lsem).wait()
        pltpu.make_async_copy(w_hbm.at[pl.ds(0, K)], wv, lsem).wait()

    def gather_start(sl):
        sv, dv, gv, wv, lsem, gsem, ssem = sl
        pltpu.async_copy(xl_hbm.at[sv], gv, gsem)

    def gather_wait(sl):
        sv, dv, gv, wv, lsem, gsem, ssem = sl
        pltpu.make_async_copy(xl_hbm.at[sv], gv, gsem).wait()

    def mul(sl):
        sv, dv, gv, wv, lsem, gsem, ssem = sl

        @plsc.parallel_loop(0, K, 1, unroll=4)
        def _mrow(e):
            for q in range(D // 32):
                uw = wv[e, pl.ds(q * 16, 16)]
                hi = jnp.uint32(0xFFFF0000)
                wa = jax.lax.bitcast_convert_type(uw << 16, jnp.float32)
                wb = jax.lax.bitcast_convert_type(uw & hi, jnp.float32)
                sl0 = pl.ds(q * 32, 16)
                sl1 = pl.ds(q * 32 + 16, 16)
                gv[e, sl0] = gv[e, sl0] * wa
                gv[e, sl1] = gv[e, sl1] * wb

    def scat_start(sl):
        sv, dv, gv, wv, lsem, gsem, ssem = sl
        pltpu.async_copy(gv, agg_sh.at[dv], ssem, add=True)

    def scat_wait(sl):
        sv, dv, gv, wv, lsem, gsem, ssem = sl
        pltpu.make_async_copy(gv, agg_sh.at[dv], ssem).wait()

    def step(j, sl_cur, sl_nxt, has_next, has_next2):
        gather_wait(sl_cur)
        mul(sl_cur)
        scat_start(sl_cur)
        if has_next:
            lin_wait(sl_nxt)
            gather_start(sl_nxt)
        scat_wait(sl_cur)
        if has_next2:
            lin_start(j + 2, sl_cur)

    lin_start(0, slots[0])
    lin_wait(slots[0])
    gather_start(slots[0])
    lin_start(1, slots[1])

    def pair(t, carry):
        j = 2 * t
        step(j, slots[0], slots[1], True, True)
        step(j + 1, slots[1], slots[0], True, True)
        return carry

    lax.fori_loop(0, (NBLK - 3) // 2, pair, 0)
    step(NBLK - 3, slots[0], slots[1], True, True)
    step(NBLK - 2, slots[1], slots[0], True, False)
    step(NBLK - 1, slots[0], slots[1], False, False)

    plsc.subcore_barrier()
    pltpu.sync_copy(agg_sh.at[pl.ds(r0, ROWS_PER_SUB)],
                    out_hbm.at[pl.ds(c * N + r0, ROWS_PER_SUB)])

    @pl.when(s == NS - 1)
    def _():
        pltpu.sync_copy(agg_sh.at[pl.ds(rtail, ROWS_TAIL)],
                        out_hbm.at[pl.ds(c * N + rtail, ROWS_TAIL)])


_sc_scatter = functools.partial(
    pl.kernel,
    out_type=jax.ShapeDtypeStruct((2 * N, D), jnp.float32),
    mesh=plsc.VectorSubcoreMesh(core_axis_name="c", subcore_axis_name="s",
                                num_cores=NC, num_subcores=NS),
    scratch_types=[
        pltpu.VMEM((K,), jnp.int32),
        pltpu.VMEM((K,), jnp.int32),
        pltpu.VMEM((K, D), jnp.float32),
        pltpu.VMEM((K, D // 2), jnp.uint32),
        pltpu.VMEM((K,), jnp.int32),
        pltpu.VMEM((K,), jnp.int32),
        pltpu.VMEM((K, D), jnp.float32),
        pltpu.VMEM((K, D // 2), jnp.uint32),
        pltpu.VMEM_SHARED((N, D), jnp.float32),
        pltpu.SemaphoreType.DMA,
        pltpu.SemaphoreType.DMA,
        pltpu.SemaphoreType.DMA,
        pltpu.SemaphoreType.DMA,
        pltpu.SemaphoreType.DMA,
        pltpu.SemaphoreType.DMA,
    ],
)(_sc_body)


def kernel(node_input, node_attr, edge_index, edge_attr, edge_scalars,
           W_sc_0, W_lin1_0, W_fc1_0, W_fc2_0, W_lin2_0,
           W_sc_1, W_lin1_1, W_fc1_1, W_fc2_1, W_lin2_1,
           W_sc_2, W_lin1_2, W_fc1_2, W_fc2_2, W_lin2_2):
    src = edge_index[0]
    dst = edge_index[1]

    inv_d = 1.0 / math.sqrt(D)
    f1_scale = 1.0 / math.sqrt(NB)
    f2_scale = 1.0 / (math.sqrt(RN) * math.sqrt(32.0))

    f1s = [W_fc1_0 * f1_scale, W_fc1_1 * f1_scale, W_fc1_2 * f1_scale]
    f2s = [W_fc2_0 * f2_scale, W_fc2_1 * f2_scale, W_fc2_2 * f2_scale]
    wscs = [W_sc_0 * inv_d, W_sc_1 * inv_d, W_sc_2 * inv_d]
    wl1s = [W_lin1_0 * inv_d, W_lin1_1 * inv_d, W_lin1_2 * inv_d]
    wl2s = [W_lin2_0 * inv_d, W_lin2_1 * inv_d, W_lin2_2 * inv_d]

    zeros = jnp.zeros((N, D), jnp.float32)

    (w0,) = _edge_weights(edge_scalars, edge_attr, f1s[:1], f2s[:1])
    wfull = [w0, None, None]

    s, xl = _pre(node_input, node_attr, wscs[0], wl1s[0])
    for i in range(3):
        aggp = _sc_scatter(xl, src, dst, wfull[i], zeros)
        if i == 0:
            wfull[1], wfull[2] = _edge_weights(edge_scalars, edge_attr,
                                               f1s[1:], f2s[1:])
        if i < 2:
            s, xl = _postpre(aggp[:N], aggp[N:], node_attr, s, wl2s[i],
                             wscs[i + 1], wl1s[i + 1])
        else:
            return _post(aggp[:N], aggp[N:], node_attr, s, wl2s[i], act=False)
